# baseline (speedup 1.0000x reference)
import jax
import jax.numpy as jnp
from jax import lax
from jax.experimental import pallas as pl
from jax.experimental.pallas import tpu as pltpu

VOCAB = 1000
EMBD = 32
BATCH = 1024
SEQ = 50


def kernel(idx, tok_table, pos_table, W, b):
  idx_t3 = idx.astype(jnp.int32).T.reshape(SEQ, 1, BATCH)
  tok_t = tok_table.T
  pos_t = pos_table.T
  w_t = W.T
  b_col = b.reshape(VOCAB, 1)

  def head(idx_ref, tok_ref, pos_ref, w_ref, b_ref, out_hbm, buf, sems):
    t = pl.program_id(0)
    slot = lax.rem(t, 4)

    @pl.when(t >= 4)
    def _():
      pltpu.make_async_copy(
          buf.at[slot], out_hbm.at[t], sems.at[slot]
      ).wait()

    tok_bf = tok_ref[...].astype(jnp.bfloat16)
    w_bf = w_ref[...].astype(jnp.bfloat16)
    onehot = (
        lax.broadcasted_iota(jnp.int32, (VOCAB, BATCH), 0) == idx_ref[0]
    ).astype(jnp.bfloat16)
    emb_t = jnp.dot(tok_bf, onehot, preferred_element_type=jnp.float32)
    et = (
        lax.broadcasted_iota(jnp.int32, (SEQ, 1), 0) == t
    ).astype(jnp.float32)
    pos_col = jnp.dot(pos_ref[...], et, preferred_element_type=jnp.float32)
    x_t = (emb_t + pos_col).astype(jnp.bfloat16)
    buf[slot] = (
        jnp.dot(w_bf, x_t, preferred_element_type=jnp.float32) + b_ref[...]
    )

    pltpu.make_async_copy(buf.at[slot], out_hbm.at[t], sems.at[slot]).start()

    @pl.when(t == SEQ - 1)
    def _():
      pltpu.make_async_copy(
          buf.at[slot], out_hbm.at[t], sems.at[slot]
      ).wait()
      for d in (1, 2, 3):
        other = lax.rem(slot + d, 4)
        pltpu.make_async_copy(
            buf.at[other], out_hbm.at[t], sems.at[other]
        ).wait()

  out = pl.pallas_call(
      head,
      grid=(SEQ,),
      in_specs=[
          pl.BlockSpec((1, 1, BATCH), lambda i: (i, 0, 0)),
          pl.BlockSpec((EMBD, VOCAB), lambda i: (0, 0)),
          pl.BlockSpec((EMBD, SEQ), lambda i: (0, 0)),
          pl.BlockSpec((VOCAB, EMBD), lambda i: (0, 0)),
          pl.BlockSpec((VOCAB, 1), lambda i: (0, 0)),
      ],
      out_specs=pl.BlockSpec(memory_space=pl.ANY),
      out_shape=jax.ShapeDtypeStruct((SEQ, VOCAB, BATCH), jnp.float32),
      scratch_shapes=[
          pltpu.VMEM((4, VOCAB, BATCH), jnp.float32),
          pltpu.SemaphoreType.DMA((4,)),
      ],
      compiler_params=pltpu.CompilerParams(
          dimension_semantics=("arbitrary",),
      ),
  )(idx_t3, tok_t, pos_t, w_t, b_col)
  return jnp.transpose(out, (2, 0, 1))


# speedup vs baseline: 1.0050x; 1.0050x over previous
import jax
import jax.numpy as jnp
from jax import lax
from jax.experimental import pallas as pl
from jax.experimental.pallas import tpu as pltpu

VOCAB = 1000
EMBD = 32
BATCH = 1024
SEQ = 50
NSLOT = 3


def kernel(idx, tok_table, pos_table, W, b):
  idx_t3 = idx.astype(jnp.int32).T.reshape(SEQ, 1, BATCH)
  tok_t = tok_table.T
  pos_t = pos_table.T
  w_t = W.T
  b_col = b.reshape(VOCAB, 1)

  def head(idx_ref, tok_ref, pos_ref, w_ref, b_ref, out_hbm, buf, sems):
    t = pl.program_id(0)
    slot = lax.rem(t, NSLOT)

    @pl.when(t >= NSLOT)
    def _():
      pltpu.make_async_copy(
          buf.at[slot], out_hbm.at[t], sems.at[slot]
      ).wait()

    tok_bf = tok_ref[...].astype(jnp.bfloat16)
    w_bf = w_ref[...].astype(jnp.bfloat16)
    onehot = (
        lax.broadcasted_iota(jnp.int32, (VOCAB, BATCH), 0) == idx_ref[0]
    ).astype(jnp.bfloat16)
    emb_t = jnp.dot(tok_bf, onehot, preferred_element_type=jnp.float32)
    et = (
        lax.broadcasted_iota(jnp.int32, (SEQ, 1), 0) == t
    ).astype(jnp.float32)
    pos_col = jnp.dot(pos_ref[...], et, preferred_element_type=jnp.float32)
    x_t = (emb_t + pos_col).astype(jnp.bfloat16)
    buf[slot] = (
        jnp.dot(w_bf, x_t, preferred_element_type=jnp.float32) + b_ref[...]
    )

    pltpu.make_async_copy(buf.at[slot], out_hbm.at[t], sems.at[slot]).start()

    @pl.when(t == SEQ - 1)
    def _():
      pltpu.make_async_copy(
          buf.at[slot], out_hbm.at[t], sems.at[slot]
      ).wait()
      for d in range(1, NSLOT):
        other = lax.rem(slot + d, NSLOT)
        pltpu.make_async_copy(
            buf.at[other], out_hbm.at[t], sems.at[other]
        ).wait()

  out = pl.pallas_call(
      head,
      grid=(SEQ,),
      in_specs=[
          pl.BlockSpec((1, 1, BATCH), lambda i: (i, 0, 0)),
          pl.BlockSpec((EMBD, VOCAB), lambda i: (0, 0)),
          pl.BlockSpec((EMBD, SEQ), lambda i: (0, 0)),
          pl.BlockSpec((VOCAB, EMBD), lambda i: (0, 0)),
          pl.BlockSpec((VOCAB, 1), lambda i: (0, 0)),
      ],
      out_specs=pl.BlockSpec(memory_space=pl.ANY),
      out_shape=jax.ShapeDtypeStruct((SEQ, VOCAB, BATCH), jnp.float32),
      scratch_shapes=[
          pltpu.VMEM((NSLOT, VOCAB, BATCH), jnp.float32),
          pltpu.SemaphoreType.DMA((NSLOT,)),
      ],
      compiler_params=pltpu.CompilerParams(
          dimension_semantics=("arbitrary",),
      ),
  )(idx_t3, tok_t, pos_t, w_t, b_col)
  return jnp.transpose(out, (2, 0, 1))
